# baseline (speedup 1.0000x reference)
import functools

import jax
import jax.numpy as jnp
from jax import lax
from jax.experimental import pallas as pl
from jax.experimental.pallas import tpu as pltpu
from jax.experimental.pallas import tpu_sc as plsc

_N = 10000
_NP = 10240
_HP1 = 256
_HP2 = 384
_E = 320000
_NB = 64
_BR = _NP // _NB
_C = 64
_T = _E + _NB * _C
_NW = 32


def _mm_body(a_ref, b_ref, o_ref, acc_ref, *, nk):
    k = pl.program_id(1)

    @pl.when(k == 0)
    def _():
        acc_ref[...] = jnp.zeros_like(acc_ref)

    acc_ref[...] += jnp.dot(a_ref[...].astype(jnp.bfloat16),
                            b_ref[...].astype(jnp.bfloat16),
                            preferred_element_type=jnp.float32)

    @pl.when(k == nk - 1)
    def _():
        o_ref[...] = acc_ref[...]


def _matmul(a, b, bm, bk):
    m, k = a.shape
    _, n = b.shape
    nk = k // bk
    return pl.pallas_call(
        functools.partial(_mm_body, nk=nk),
        grid=(m // bm, nk),
        in_specs=[
            pl.BlockSpec((bm, bk), lambda i, j: (i, j)),
            pl.BlockSpec((bk, n), lambda i, j: (j, 0)),
        ],
        out_specs=pl.BlockSpec((bm, n), lambda i, j: (i, 0)),
        scratch_shapes=[pltpu.VMEM((bm, n), jnp.float32)],
        out_shape=jax.ShapeDtypeStruct((m, n), jnp.float32),
    )(a, b)


def _gru_body(agg_ref, x_ref, wih_ref, whh_ref, bih_ref, bhh_ref, o_ref,
              *, h, relu):
    gi = jnp.dot(agg_ref[...].astype(jnp.bfloat16),
                 wih_ref[...].astype(jnp.bfloat16),
                 preferred_element_type=jnp.float32) + bih_ref[...]
    gh = jnp.dot(x_ref[...].astype(jnp.bfloat16),
                 whh_ref[...].astype(jnp.bfloat16),
                 preferred_element_type=jnp.float32) + bhh_ref[...]
    r = jax.nn.sigmoid(gi[:, :h] + gh[:, :h])
    z = jax.nn.sigmoid(gi[:, h:2 * h] + gh[:, h:2 * h])
    c = jnp.tanh(gi[:, 2 * h:] + r * gh[:, 2 * h:])
    out = (1.0 - z) * c + z * x_ref[...]
    if relu:
        out = jnp.maximum(out, 0.0)
    o_ref[...] = out


def _gru(agg, x, wihT, whhT, bih, bhh, relu, bm=2048):
    n, h = x.shape
    return pl.pallas_call(
        functools.partial(_gru_body, h=h, relu=relu),
        grid=(n // bm,),
        in_specs=[
            pl.BlockSpec((bm, h), lambda i: (i, 0)),
            pl.BlockSpec((bm, h), lambda i: (i, 0)),
            pl.BlockSpec((h, 3 * h), lambda i: (0, 0)),
            pl.BlockSpec((h, 3 * h), lambda i: (0, 0)),
            pl.BlockSpec((1, 3 * h), lambda i: (0, 0)),
            pl.BlockSpec((1, 3 * h), lambda i: (0, 0)),
        ],
        out_specs=pl.BlockSpec((bm, h), lambda i: (i, 0)),
        out_shape=jax.ShapeDtypeStruct((n, h), jnp.float32),
    )(agg, x, wihT, whhT, bih, bhh)


_CH = 128
_TCH = -(-_E // (16 * _CH)) * _CH
_EP = 16 * _TCH
_NCH = _TCH // _CH


def _make_sc_agg(hp):
    g = hp // 128
    mesh = plsc.VectorSubcoreMesh(core_axis_name="c", subcore_axis_name="s",
                                  num_cores=1)

    @functools.partial(
        pl.kernel,
        mesh=mesh,
        out_type=jax.ShapeDtypeStruct((g * _NP, 128), jnp.float32),
        scratch_types=[
            pltpu.VMEM_SHARED((_NP + 8, 128), jnp.float32),
            pltpu.VMEM((2, _CH), jnp.int32),
            pltpu.VMEM((2, _CH), jnp.int32),
            pltpu.VMEM((2, _CH, 128), jnp.float32),
            pltpu.SemaphoreType.DMA,
            pltpu.SemaphoreType.DMA,
        ],
    )
    def k(m_hbm, src_hbm, dst_hbm, z_hbm, out_hbm,
          acc_sh, gi_v, sc_v, rows_v, sem_g, sem_s):
        sid = lax.axis_index("s")

        def gather(p):
            return pltpu.async_copy(m_hbm.at[gi_v.at[p]], rows_v.at[p],
                                    sem_g)

        def scatter(p):
            return pltpu.async_copy(rows_v.at[p], acc_sh.at[sc_v.at[p]],
                                    sem_s, add=True)

        def wait_scatter(p):
            pltpu.make_async_copy(rows_v.at[p], acc_sh.at[sc_v.at[p]],
                                  sem_s).wait()

        for grp in range(g):
            pltpu.sync_copy(z_hbm,
                            acc_sh.at[pl.ds(sid * (_NP // 16), _NP // 16)])

            @pl.when(sid == 0)
            def _():
                pltpu.sync_copy(z_hbm.at[pl.ds(0, 8)],
                                acc_sh.at[pl.ds(_NP, 8)])

            plsc.subcore_barrier()

            def chunk(kk, carry):
                p = kk % 2
                base = sid * _TCH + kk * _CH
                pltpu.sync_copy(dst_hbm.at[pl.ds(base, _CH)], sc_v.at[p])
                pltpu.sync_copy(src_hbm.at[pl.ds(base, _CH)], gi_v.at[p])
                if grp:
                    for j in range(_CH // 16):
                        gi_v[p, pl.ds(j * 16, 16)] = (
                            gi_v[p, pl.ds(j * 16, 16)] + grp * _NP)
                gather(p)
                pltpu.make_async_copy(m_hbm.at[gi_v.at[p]],
                                      rows_v.at[p], sem_g).wait()

                @pl.when(kk > 0)
                def _():
                    wait_scatter(1 - p)

                scatter(p)
                return carry

            lax.fori_loop(0, _NCH, chunk, 0)
            wait_scatter((_NCH - 1) % 2)
            plsc.subcore_barrier()
            pltpu.sync_copy(
                acc_sh.at[pl.ds(sid * (_NP // 16), _NP // 16)],
                out_hbm.at[pl.ds(grp * _NP + sid * (_NP // 16),
                                 _NP // 16)])
            plsc.subcore_barrier()

    return k
    mesh = plsc.VectorSubcoreMesh(core_axis_name="c", subcore_axis_name="s")

    @functools.partial(
        pl.kernel,
        mesh=mesh,
        out_type=jax.ShapeDtypeStruct((_NP, hp), jnp.float32),
        scratch_types=[
            pltpu.VMEM_SHARED((_SROWS + 8, hp), jnp.float32),
            pltpu.VMEM((_BR, hp), jnp.float32),
            pltpu.VMEM((_C, hp), jnp.float32),
            pltpu.VMEM((_C,), jnp.int32),
            pltpu.VMEM((_C,), jnp.int32),
            pltpu.VMEM((_NB,), jnp.int32),
            pltpu.VMEM((_NB,), jnp.int32),
            pltpu.SemaphoreType.DMA,
        ],
    )
    def k(m_hbm, srcp_hbm, ldp_hbm, s0_hbm, nch_hbm, out_hbm,
          acc_sh, zer_v, rows_v, idx_v, ld_v, s0_v, nch_v, sem):
        cid = lax.axis_index("c")
        sid = lax.axis_index("s")
        pltpu.sync_copy(s0_hbm, s0_v)
        pltpu.sync_copy(nch_hbm, nch_v)
        lane = lax.iota(jnp.int32, 16)
        zeros16 = jnp.zeros((16,), jnp.float32)

        def zrow(i, carry):
            for j in range(hp // 16):
                zer_v[i, pl.ds(j * 16, 16)] = zeros16
            return carry

        lax.fori_loop(0, _BR, zrow, 0)

        for r in range(_NB // _NW):
            b = cid * 16 + sid + _NW * r
            lb = sid + 16 * r
            base_row = lb * _BR
            pltpu.sync_copy(zer_v, acc_sh.at[pl.ds(base_row, _BR)])

            g16 = (cid + 2 * r) * 16

            def scalar_at(vref):
                v = jnp.where(lane == sid, vref[pl.ds(g16, 16)], 0)
                total = jnp.int32(0)
                for kb in range(13):
                    bit = jnp.any((v & (1 << kb)) != 0)
                    total = total + jnp.where(bit, jnp.int32(1 << kb),
                                              jnp.int32(0))
                return total

            s0 = scalar_at(s0_v)
            nch = scalar_at(nch_v)

            def chunk(c, carry):
                cbase = (s0 + c) * _C
                pltpu.sync_copy(srcp_hbm.at[pl.ds(cbase, _C)], idx_v)
                pltpu.sync_copy(ldp_hbm.at[pl.ds(cbase, _C)], ld_v)
                for j in range(_C // 16):
                    v = ld_v[pl.ds(j * 16, 16)]
                    ld_v[pl.ds(j * 16, 16)] = jnp.where(
                        v == _BR, _DUMMY, v + base_row)
                pltpu.async_copy(m_hbm.at[idx_v], rows_v, sem).wait()
                pltpu.sync_copy(rows_v, acc_sh.at[ld_v], add=True)
                return carry

            lax.fori_loop(0, nch, chunk, 0)
            pltpu.sync_copy(acc_sh.at[pl.ds(base_row, _BR)],
                            out_hbm.at[pl.ds(b * _BR, _BR)])

    return k


def _pad_sq(w, hp):
    h = w.shape[0]
    return jnp.pad(w, ((0, hp - h), (0, hp - h)))


def _pad_gates(wih, bih, hp):
    h = wih.shape[1]
    wt = wih.T
    parts = [jnp.pad(wt[:, g * h:(g + 1) * h], ((0, hp - h), (0, hp - h)))
             for g in range(3)]
    bparts = [jnp.pad(bih[g * h:(g + 1) * h], (0, hp - h)) for g in range(3)]
    return (jnp.concatenate(parts, axis=1),
            jnp.concatenate(bparts)[None, :])


def _gated_layer(h, srcp, dstp, W, Wih, Whh, bih, bhh, hp, relu_last):
    g = hp // 128
    sc_agg = _make_sc_agg(hp)
    zeros = jnp.zeros((_NP // 16, 128), jnp.float32)
    wihT, bihp = _pad_gates(Wih, bih, hp)
    whhT, bhhp = _pad_gates(Whh, bhh, hp)
    L = W.shape[0]
    for l in range(L):
        wl = _pad_sq(W[l], hp)
        m = _matmul(h, wl, bm=2048, bk=hp)
        m2 = m.reshape(_NP, g, 128).transpose(1, 0, 2).reshape(g * _NP, 128)
        agg2 = sc_agg(m2, srcp, dstp, zeros)
        agg = agg2.reshape(g, _NP, 128).transpose(1, 0, 2).reshape(_NP, hp)
        gi = _matmul(agg, wihT, bm=2048, bk=hp) + bihp
        gh = _matmul(h, whhT, bm=2048, bk=hp) + bhhp
        r = jax.nn.sigmoid(gi[:, :hp] + gh[:, :hp])
        z = jax.nn.sigmoid(gi[:, hp:2 * hp] + gh[:, hp:2 * hp])
        c = jnp.tanh(gi[:, 2 * hp:] + r * gh[:, 2 * hp:])
        h = (1.0 - z) * c + z * h
        if relu_last and l == L - 1:
            h = jax.nn.relu(h)
    return h


def kernel(x, edge_index, W1, Wih1, Whh1, bih1, bhh1,
           W2, Wih2, Whh2, bih2, bhh2):
    src = edge_index[0]
    dst = edge_index[1]
    order = jnp.argsort(dst, stable=True)
    ds = dst[order]
    ss = src[order]
    iota = jnp.arange(_E, dtype=jnp.int32)
    rank = iota - jnp.searchsorted(ds, ds, side="left").astype(jnp.int32)
    tile = iota // (_E // 16)
    key = tile * 32768 + jnp.minimum(rank, 32767)
    order2 = jnp.argsort(key, stable=True)
    pos = iota + (_TCH - _E // 16) * tile
    srcp = jnp.full((_EP,), _NP - 1, jnp.int32).at[pos].set(ss[order2])
    dstp = jnp.full((_EP,), _NP, jnp.int32).at[pos].set(ds[order2])
    h = jnp.pad(x, ((0, _NP - _N), (0, _HP1 - x.shape[1])))
    h = _gated_layer(h, srcp, dstp, W1, Wih1, Whh1, bih1, bhh1, _HP1,
                     relu_last=True)
    h = jnp.pad(h, ((0, 0), (0, _HP2 - _HP1)))
    h = _gated_layer(h, srcp, dstp, W2, Wih2, Whh2, bih2, bhh2, _HP2,
                     relu_last=False)
    return h[:_N, :300]
